# baseline (speedup 1.0000x reference)
import functools

import jax
import jax.numpy as jnp
from jax import lax
from jax.experimental import pallas as pl
from jax.experimental.pallas import tpu as pltpu
from jax.experimental.pallas import tpu_sc as plsc

B, L, D, H = 4096, 200, 64, 128
NC, NS = 2, 16
NW = NC * NS
RPW = B // NW
NCHUNK = 2
CHUNK = L // NCHUNK
NLANE = 16
NCG = D // NLANE

_mesh = plsc.VectorSubcoreMesh(
    core_axis_name="c", subcore_axis_name="s", num_cores=NC, num_subcores=NS
)


@functools.partial(
    pl.kernel,
    out_type=jax.ShapeDtypeStruct((B, D), jnp.float32),
    mesh=_mesh,
    scratch_types=[
        pltpu.VMEM((RPW, NCHUNK, CHUNK), jnp.int32),
        pltpu.VMEM((NCHUNK, CHUNK, D), jnp.float32),
        pltpu.VMEM((RPW, D), jnp.float32),
        pltpu.SemaphoreType.DMA,
    ],
    compiler_params=pltpu.CompilerParams(use_tc_tiling_on_sc=False),
)
def _pool_kernel(x_hbm, table_hbm, out_hbm, idx_v, rows_v, out_v, sem):
    wid = lax.axis_index("s") * NC + lax.axis_index("c")
    base = wid * RPW
    pltpu.sync_copy(x_hbm.at[pl.ds(base, RPW)], idx_v)

    def row_body(b, carry):
        cps = [
            pltpu.async_copy(table_hbm.at[idx_v.at[b, j]], rows_v.at[j], sem)
            for j in range(NCHUNK)
        ]
        for cp in cps:
            cp.wait()

        def red_body(r, accs):
            res = []
            for c in range(NCG):
                a = accs[c]
                for j in range(NCHUNK):
                    a = jnp.maximum(a, rows_v[j, r, pl.ds(c * NLANE, NLANE)])
                res.append(a)
            return tuple(res)

        init = tuple(
            jnp.full((NLANE,), -jnp.inf, jnp.float32) for _ in range(NCG)
        )
        accs = lax.fori_loop(0, CHUNK, red_body, init)
        for c in range(NCG):
            out_v[b, pl.ds(c * NLANE, NLANE)] = accs[c]
        return carry

    lax.fori_loop(0, RPW, row_body, 0)
    pltpu.sync_copy(out_v, out_hbm.at[pl.ds(base, RPW)])


def _head_body(p_ref, w_ref, b_ref, o_ref):
    h = lax.dot_general(
        p_ref[...], w_ref[...], (((1,), (1,)), ((), ())),
        preferred_element_type=jnp.float32,
    )
    h = h + b_ref[...]
    s = jnp.sum(h * h, axis=1, keepdims=True)
    o_ref[...] = h * lax.rsqrt(jnp.maximum(s, 1e-24))


def kernel(x, embed_table, W, b):
    x3 = x.astype(jnp.int32).reshape(B, NCHUNK, CHUNK)
    pooled = _pool_kernel(x3, embed_table)
    out = pl.pallas_call(
        _head_body,
        out_shape=jax.ShapeDtypeStruct((B, H), jnp.float32),
    )(pooled, W, b.reshape(1, H))
    return out


# speedup vs baseline: 1.1340x; 1.1340x over previous
import functools

import jax
import jax.numpy as jnp
from jax import lax
from jax.experimental import pallas as pl
from jax.experimental.pallas import tpu as pltpu
from jax.experimental.pallas import tpu_sc as plsc

B, L, D, H = 4096, 200, 64, 128
NC, NS = 2, 16
NW = NC * NS
RPW = B // NW
NCHUNK = 2
CHUNK = L // NCHUNK
NLANE = 16
NCG = D // NLANE

_mesh = plsc.VectorSubcoreMesh(
    core_axis_name="c", subcore_axis_name="s", num_cores=NC, num_subcores=NS
)


UNROLL = 4


@functools.partial(
    pl.kernel,
    out_type=jax.ShapeDtypeStruct((B, D), jnp.float32),
    mesh=_mesh,
    scratch_types=[
        pltpu.VMEM((RPW, NCHUNK, CHUNK), jnp.int32),
        pltpu.VMEM((2, NCHUNK, CHUNK, D), jnp.float32),
        pltpu.VMEM((RPW, D), jnp.float32),
        pltpu.SemaphoreType.DMA,
        pltpu.SemaphoreType.DMA,
    ],
    compiler_params=pltpu.CompilerParams(use_tc_tiling_on_sc=False),
)
def _pool_kernel(x_hbm, table_hbm, out_hbm, idx_v, rows_v, out_v, sem0, sem1):
    wid = lax.axis_index("s") * NC + lax.axis_index("c")
    base = wid * RPW
    sems = (sem0, sem1)
    pltpu.sync_copy(x_hbm.at[pl.ds(base, RPW)], idx_v)

    def start(b, p):
        for j in range(NCHUNK):
            pltpu.async_copy(
                table_hbm.at[idx_v.at[b, j]], rows_v.at[p, j], sems[p]
            )

    def wait(b, p):
        for j in range(NCHUNK):
            pltpu.make_async_copy(
                table_hbm.at[idx_v.at[b, j]], rows_v.at[p, j], sems[p]
            ).wait()

    def reduce(b, p):
        def red_body(t, accs):
            res = list(accs)
            for u in range(UNROLL):
                r = t * UNROLL + u
                for j in range(NCHUNK):
                    for c in range(NCG):
                        res[c] = jnp.maximum(
                            res[c], rows_v[p, j, r, pl.ds(c * NLANE, NLANE)]
                        )
            return tuple(res)

        init = tuple(
            jnp.full((NLANE,), -jnp.inf, jnp.float32) for _ in range(NCG)
        )
        accs = lax.fori_loop(0, CHUNK // UNROLL, red_body, init)
        for c in range(NCG):
            out_v[b, pl.ds(c * NLANE, NLANE)] = accs[c]

    start(0, 0)
    start(1, 1)

    def row_body(i, carry):
        for p in range(2):
            b = 2 * i + p
            wait(b, p)
            reduce(b, p)
            start(b + 2, p)
        return carry

    lax.fori_loop(0, RPW // 2 - 1, row_body, 0)
    for p in range(2):
        b = RPW - 2 + p
        wait(b, p)
        reduce(b, p)
    pltpu.sync_copy(out_v, out_hbm.at[pl.ds(base, RPW)])


def _head_body(p_ref, w_ref, b_ref, o_ref):
    h = lax.dot_general(
        p_ref[...], w_ref[...], (((1,), (1,)), ((), ())),
        preferred_element_type=jnp.float32,
    )
    h = h + b_ref[...]
    s = jnp.sum(h * h, axis=1, keepdims=True)
    o_ref[...] = h * lax.rsqrt(jnp.maximum(s, 1e-24))


def kernel(x, embed_table, W, b):
    x3 = x.astype(jnp.int32).reshape(B, NCHUNK, CHUNK)
    pooled = _pool_kernel(x3, embed_table)
    out = pl.pallas_call(
        _head_body,
        out_shape=jax.ShapeDtypeStruct((B, H), jnp.float32),
    )(pooled, W, b.reshape(1, H))
    return out
